# baseline (speedup 1.0000x reference)
import functools
import math

import jax
import jax.numpy as jnp
from jax.experimental import pallas as pl
from jax.experimental.pallas import tpu as pltpu

S = 4096
IN = 1024
H = 16
D = 64
ED = H * D
C = 100
K = 8
SB = 512


def _ln(x, g, b, eps=1e-5):
    m = jnp.mean(x, axis=-1, keepdims=True)
    d = x - m
    v = jnp.mean(d * d, axis=-1, keepdims=True)
    return d * jax.lax.rsqrt(v + eps) * g + b


def _c_kernel(x_ref, wih_ref, bih_ref, c_ref):
    c_ref[...] = jnp.dot(x_ref[...], wih_ref[...],
                         preferred_element_type=jnp.float32) + bih_ref[...]


def _topk_kernel(c_ref, idx_ref):
    cv = c_ref[...]
    iota = jax.lax.broadcasted_iota(jnp.int32, (S, C), 0)
    for k in range(K):
        m = jnp.max(cv, axis=0, keepdims=True)
        hit = cv >= m
        idx = jnp.min(jnp.where(hit, iota, S), axis=0)
        idx_ref[k, :] = idx
        cv = jnp.where(iota == idx[None, :], -jnp.inf, cv)


def _kproj_kernel(idx_ref, x_ref, wk_ref, bk_ref, gk_ref, bbk_ref, k_ref, acc_ref):
    j = pl.program_id(0)
    idx_j = idx_ref[pl.ds(j, 1), :]
    onehot = (jax.lax.broadcasted_iota(jnp.int32, (C, S), 1)
              == jnp.reshape(idx_j, (C, 1))).astype(jnp.float32)
    g = jnp.dot(onehot, x_ref[...], preferred_element_type=jnp.float32)
    contrib = jnp.dot(g, wk_ref[...], preferred_element_type=jnp.float32)

    @pl.when(j == 0)
    def _():
        acc_ref[...] = contrib

    @pl.when(j > 0)
    def _():
        acc_ref[...] += contrib

    @pl.when(j == K - 1)
    def _():
        pre = acc_ref[...] + bk_ref[...]
        kf = _ln(pre, gk_ref[...], bbk_ref[...])
        k_ref[...] = jnp.transpose(kf.reshape(C, H, D), (1, 0, 2))


def _attn_kernel(x_ref, wq_ref, bq_ref, gq_ref, bbq_ref,
                 wv_ref, bv_ref, gv_ref, bbv_ref, k_ref,
                 attn_ref, val_ref):
    i = pl.program_id(0)
    xb = x_ref[...]
    qp = jnp.dot(xb, wq_ref[...], preferred_element_type=jnp.float32) + bq_ref[...]
    qt = jnp.transpose(_ln(qp, gq_ref[...], bbq_ref[...]).reshape(SB, H, D), (1, 0, 2))
    vp = jnp.dot(xb, wv_ref[...], preferred_element_type=jnp.float32) + bv_ref[...]
    vt = jnp.transpose(_ln(vp, gv_ref[...], bbv_ref[...]).reshape(SB, H, D), (1, 0, 2))
    scale = 1.0 / math.sqrt(D)
    for h in range(H):
        qh = qt[h]
        kh = k_ref[h]
        logits = jax.lax.dot_general(qh, kh, (((1,), (1,)), ((), ())),
                                     preferred_element_type=jnp.float32) * scale
        mx = jnp.max(logits, axis=1, keepdims=True)
        e = jnp.exp(logits - mx)
        a = e / jnp.sum(e, axis=1, keepdims=True)
        attn_ref[h, :, :] = a
        contrib = jax.lax.dot_general(a, vt[h], (((0,), (0,)), ((), ())),
                                      preferred_element_type=jnp.float32)

        @pl.when(i == 0)
        def _():
            val_ref[h, :, :] = contrib

        @pl.when(i > 0)
        def _():
            val_ref[h, :, :] += contrib


def _out_kernel(val_ref, wp_ref, bp_ref, go_ref, bbo_ref, o_ref):
    vals = val_ref[...]
    acc = jnp.zeros((C, IN), jnp.float32)
    for h in range(H):
        acc += jnp.dot(vals[h], wp_ref[pl.ds(h * D, D), :],
                       preferred_element_type=jnp.float32)
    pre = acc + bp_ref[...]
    o_ref[...] = _ln(pre, go_ref[...], bbo_ref[...])


def kernel(x, W_ih, b_ih, W_k, b_k, g_k, bb_k, W_q, b_q, g_q, bb_q,
           W_v, b_v, g_v, bb_v, W_p, b_p, g_o, bb_o):
    f32 = jnp.float32
    b_ih2 = b_ih.reshape(1, C)
    b_q2, g_q2, bb_q2 = b_q.reshape(1, ED), g_q.reshape(1, ED), bb_q.reshape(1, ED)
    b_v2, g_v2, bb_v2 = b_v.reshape(1, ED), g_v.reshape(1, ED), bb_v.reshape(1, ED)
    b_k2, g_k2, bb_k2 = b_k.reshape(1, ED), g_k.reshape(1, ED), bb_k.reshape(1, ED)
    b_p2, g_o2, bb_o2 = b_p.reshape(1, IN), g_o.reshape(1, IN), bb_o.reshape(1, IN)

    full = lambda shape: pl.BlockSpec(shape, lambda *_: tuple(0 for _ in shape))

    nsb = S // SB

    c = pl.pallas_call(
        _c_kernel,
        grid=(nsb,),
        in_specs=[
            pl.BlockSpec((SB, IN), lambda i: (i, 0)),
            full((IN, C)), full((1, C)),
        ],
        out_specs=pl.BlockSpec((SB, C), lambda i: (i, 0)),
        out_shape=jax.ShapeDtypeStruct((S, C), f32),
    )(x, W_ih, b_ih2)

    topk_idx = pl.pallas_call(
        _topk_kernel,
        out_shape=jax.ShapeDtypeStruct((K, C), jnp.int32),
    )(c)

    k_ = pl.pallas_call(
        _kproj_kernel,
        grid=(K,),
        in_specs=[
            full((K, C)),
            full((S, IN)),
            pl.BlockSpec((IN, ED), lambda j: (j, 0)),
            full((1, ED)), full((1, ED)), full((1, ED)),
        ],
        out_specs=pl.BlockSpec((H, C, D), lambda j: (0, 0, 0)),
        out_shape=jax.ShapeDtypeStruct((H, C, D), f32),
        scratch_shapes=[pltpu.VMEM((C, ED), f32)],
    )(topk_idx, x, W_k, b_k2, g_k2, bb_k2)

    attn, values = pl.pallas_call(
        _attn_kernel,
        grid=(nsb,),
        in_specs=[
            pl.BlockSpec((SB, IN), lambda i: (i, 0)),
            full((IN, ED)), full((1, ED)), full((1, ED)), full((1, ED)),
            full((IN, ED)), full((1, ED)), full((1, ED)), full((1, ED)),
            full((H, C, D)),
        ],
        out_specs=[
            pl.BlockSpec((H, SB, C), lambda i: (0, i, 0)),
            full((H, C, D)),
        ],
        out_shape=[
            jax.ShapeDtypeStruct((H, S, C), f32),
            jax.ShapeDtypeStruct((H, C, D), f32),
        ],
    )(x, W_q, b_q2, g_q2, bb_q2, W_v, b_v2, g_v2, bb_v2, k_)

    o = pl.pallas_call(
        _out_kernel,
        out_shape=jax.ShapeDtypeStruct((C, IN), f32),
    )(values, W_p, b_p2, g_o2, bb_o2)

    return (o, c, attn, topk_idx)


# speedup vs baseline: 1.2093x; 1.2093x over previous
import functools
import math

import jax
import jax.numpy as jnp
from jax.experimental import pallas as pl
from jax.experimental.pallas import tpu as pltpu

S = 4096
IN = 1024
H = 16
D = 64
ED = H * D
C = 100
K = 8
SB = 512

bf16 = jnp.bfloat16


def _ln(x, g, b, eps=1e-5):
    m = jnp.mean(x, axis=-1, keepdims=True)
    d = x - m
    v = jnp.mean(d * d, axis=-1, keepdims=True)
    return d * jax.lax.rsqrt(v + eps) * g + b


def _bdot(a, b):
    return jnp.dot(a.astype(bf16), b.astype(bf16), preferred_element_type=jnp.float32)


def _proj_kernel(x_ref, wih_ref, bih_ref, wq_ref, bq_ref, gq_ref, bbq_ref,
                 wv_ref, bv_ref, gv_ref, bbv_ref, c_ref, q_ref, v_ref):
    xb = x_ref[...]
    c_ref[...] = jnp.dot(xb, wih_ref[...], preferred_element_type=jnp.float32) + bih_ref[...]
    qp = _bdot(xb, wq_ref[...]) + bq_ref[...]
    qln = _ln(qp, gq_ref[...], bbq_ref[...])
    q_ref[...] = jnp.transpose(qln.reshape(SB, H, D), (1, 0, 2))
    vp = _bdot(xb, wv_ref[...]) + bv_ref[...]
    vln = _ln(vp, gv_ref[...], bbv_ref[...])
    v_ref[...] = jnp.transpose(vln.reshape(SB, H, D), (1, 0, 2))


def _topk_kernel(c_ref, idx_ref):
    cv = c_ref[...]
    iota = jax.lax.broadcasted_iota(jnp.int32, (S, C), 0)
    for k in range(K):
        m = jnp.max(cv, axis=0, keepdims=True)
        hit = cv >= m
        idx = jnp.min(jnp.where(hit, iota, S), axis=0)
        idx_ref[k, :] = idx
        cv = jnp.where(iota == idx[None, :], -jnp.inf, cv)


def _kproj_kernel(idx_ref, x_ref, wk_ref, bk_ref, gk_ref, bbk_ref, k_ref, acc_ref):
    j = pl.program_id(0)
    idx_j = idx_ref[pl.ds(j, 1), :]
    onehot = (jax.lax.broadcasted_iota(jnp.int32, (C, S), 1)
              == jnp.reshape(idx_j, (C, 1))).astype(bf16)
    g = jnp.dot(onehot, x_ref[...].astype(bf16),
                preferred_element_type=jnp.float32)
    contrib = _bdot(g, wk_ref[...])

    @pl.when(j == 0)
    def _():
        acc_ref[...] = contrib

    @pl.when(j > 0)
    def _():
        acc_ref[...] += contrib

    @pl.when(j == K - 1)
    def _():
        pre = acc_ref[...] + bk_ref[...]
        kf = _ln(pre, gk_ref[...], bbk_ref[...])
        k_ref[...] = jnp.transpose(kf.reshape(C, H, D), (1, 0, 2))


def _attn_kernel(q_ref, k_ref, v_ref, attn_ref, val_ref):
    qb = q_ref[0]
    kb = k_ref[0]
    logits = jax.lax.dot_general(qb, kb, (((1,), (1,)), ((), ())),
                                 preferred_element_type=jnp.float32)
    logits = logits * (1.0 / math.sqrt(D))
    mx = jnp.max(logits, axis=1, keepdims=True)
    e = jnp.exp(logits - mx)
    a = e / jnp.sum(e, axis=1, keepdims=True)
    attn_ref[0, :, :] = a
    vb = v_ref[0]
    contrib = jax.lax.dot_general(a, vb, (((0,), (0,)), ((), ())),
                                  preferred_element_type=jnp.float32)
    val_ref[0, :, :] = contrib


def _out_kernel(val_ref, wp_ref, bp_ref, go_ref, bbo_ref, o_ref):
    vals = val_ref[...]
    acc = jnp.zeros((C, IN), jnp.float32)
    for h in range(H):
        acc += jnp.dot(vals[h], wp_ref[pl.ds(h * D, D), :],
                       preferred_element_type=jnp.float32)
    pre = acc + bp_ref[...]
    o_ref[...] = _ln(pre, go_ref[...], bbo_ref[...])


def kernel(x, W_ih, b_ih, W_k, b_k, g_k, bb_k, W_q, b_q, g_q, bb_q,
           W_v, b_v, g_v, bb_v, W_p, b_p, g_o, bb_o):
    f32 = jnp.float32
    b_ih2 = b_ih.reshape(1, C)
    b_q2, g_q2, bb_q2 = b_q.reshape(1, ED), g_q.reshape(1, ED), bb_q.reshape(1, ED)
    b_v2, g_v2, bb_v2 = b_v.reshape(1, ED), g_v.reshape(1, ED), bb_v.reshape(1, ED)
    b_k2, g_k2, bb_k2 = b_k.reshape(1, ED), g_k.reshape(1, ED), bb_k.reshape(1, ED)
    b_p2, g_o2, bb_o2 = b_p.reshape(1, IN), g_o.reshape(1, IN), bb_o.reshape(1, IN)

    full = lambda shape: pl.BlockSpec(shape, lambda *_: tuple(0 for _ in shape))

    nsb = S // SB
    c, q_, v_ = pl.pallas_call(
        _proj_kernel,
        grid=(nsb,),
        in_specs=[
            pl.BlockSpec((SB, IN), lambda i: (i, 0)),
            full((IN, C)), full((1, C)),
            full((IN, ED)), full((1, ED)), full((1, ED)), full((1, ED)),
            full((IN, ED)), full((1, ED)), full((1, ED)), full((1, ED)),
        ],
        out_specs=[
            pl.BlockSpec((SB, C), lambda i: (i, 0)),
            pl.BlockSpec((H, SB, D), lambda i: (0, i, 0)),
            pl.BlockSpec((H, SB, D), lambda i: (0, i, 0)),
        ],
        out_shape=[
            jax.ShapeDtypeStruct((S, C), f32),
            jax.ShapeDtypeStruct((H, S, D), f32),
            jax.ShapeDtypeStruct((H, S, D), f32),
        ],
    )(x, W_ih, b_ih2, W_q, b_q2, g_q2, bb_q2, W_v, b_v2, g_v2, bb_v2)

    topk_idx = pl.pallas_call(
        _topk_kernel,
        out_shape=jax.ShapeDtypeStruct((K, C), jnp.int32),
    )(c)

    k_ = pl.pallas_call(
        _kproj_kernel,
        grid=(K,),
        in_specs=[
            full((K, C)),
            full((S, IN)),
            pl.BlockSpec((IN, ED), lambda j: (j, 0)),
            full((1, ED)), full((1, ED)), full((1, ED)),
        ],
        out_specs=pl.BlockSpec((H, C, D), lambda j: (0, 0, 0)),
        out_shape=jax.ShapeDtypeStruct((H, C, D), f32),
        scratch_shapes=[pltpu.VMEM((C, ED), f32)],
    )(topk_idx, x, W_k, b_k2, g_k2, bb_k2)

    attn, values = pl.pallas_call(
        _attn_kernel,
        grid=(H,),
        in_specs=[
            pl.BlockSpec((1, S, D), lambda h: (h, 0, 0)),
            pl.BlockSpec((1, C, D), lambda h: (h, 0, 0)),
            pl.BlockSpec((1, S, D), lambda h: (h, 0, 0)),
        ],
        out_specs=[
            pl.BlockSpec((1, S, C), lambda h: (h, 0, 0)),
            pl.BlockSpec((1, C, D), lambda h: (h, 0, 0)),
        ],
        out_shape=[
            jax.ShapeDtypeStruct((H, S, C), f32),
            jax.ShapeDtypeStruct((H, C, D), f32),
        ],
    )(q_, k_, v_)

    o = pl.pallas_call(
        _out_kernel,
        out_shape=jax.ShapeDtypeStruct((C, IN), f32),
    )(values, W_p, b_p2, g_o2, bb_o2)

    return (o, c, attn, topk_idx)


# speedup vs baseline: 1.2818x; 1.0600x over previous
import functools
import math

import jax
import jax.numpy as jnp
from jax.experimental import pallas as pl
from jax.experimental.pallas import tpu as pltpu

S = 4096
IN = 1024
H = 16
D = 64
ED = H * D
C = 100
K = 8
SB = 512

bf16 = jnp.bfloat16


def _ln(x, g, b, eps=1e-5):
    m = jnp.mean(x, axis=-1, keepdims=True)
    d = x - m
    v = jnp.mean(d * d, axis=-1, keepdims=True)
    return d * jax.lax.rsqrt(v + eps) * g + b


def _bdot(a, b):
    return jnp.dot(a.astype(bf16), b.astype(bf16), preferred_element_type=jnp.float32)


def _proj_kernel(x_ref, wih_ref, bih_ref, wq_ref, bq_ref, gq_ref, bbq_ref,
                 wv_ref, bv_ref, gv_ref, bbv_ref,
                 c_ref, q_ref, v_ref, idx_ref, cacc_ref):
    i = pl.program_id(0)
    xb = x_ref[...]
    cb = jnp.dot(xb, wih_ref[...], preferred_element_type=jnp.float32) + bih_ref[...]
    c_ref[...] = cb
    cacc_ref[pl.ds(i * SB, SB), :] = cb
    qp = _bdot(xb, wq_ref[...]) + bq_ref[...]
    qln = _ln(qp, gq_ref[...], bbq_ref[...]).astype(bf16)
    q_ref[...] = jnp.transpose(qln.reshape(SB, H, D), (1, 0, 2))
    vp = _bdot(xb, wv_ref[...]) + bv_ref[...]
    vln = _ln(vp, gv_ref[...], bbv_ref[...]).astype(bf16)
    v_ref[...] = jnp.transpose(vln.reshape(SB, H, D), (1, 0, 2))

    @pl.when(i == (S // SB) - 1)
    def _():
        cv = cacc_ref[...]
        iota = jax.lax.broadcasted_iota(jnp.int32, (S, C), 0)
        for k in range(K):
            m = jnp.max(cv, axis=0, keepdims=True)
            idx = jnp.min(jnp.where(cv >= m, iota, S), axis=0)
            idx_ref[k, :] = idx
            cv = jnp.where(iota == idx[None, :], -jnp.inf, cv)


def _kproj_kernel(idx_ref, x_ref, wk_ref, bk_ref, gk_ref, bbk_ref, k_ref, acc_ref):
    j = pl.program_id(0)
    idx_j = idx_ref[pl.ds(j, 1), :]
    onehot = (jax.lax.broadcasted_iota(jnp.int32, (C, S), 1)
              == jnp.reshape(idx_j, (C, 1))).astype(bf16)
    g = jnp.dot(onehot, x_ref[...].astype(bf16),
                preferred_element_type=jnp.float32)
    contrib = _bdot(g, wk_ref[...])

    @pl.when(j == 0)
    def _():
        acc_ref[...] = contrib

    @pl.when(j > 0)
    def _():
        acc_ref[...] += contrib

    @pl.when(j == K - 1)
    def _():
        pre = acc_ref[...] + bk_ref[...]
        kf = _ln(pre, gk_ref[...], bbk_ref[...]).astype(bf16)
        k_ref[...] = jnp.transpose(kf.reshape(C, H, D), (1, 0, 2))


def _attn_kernel(q_ref, k_ref, v_ref, attn_ref, val_ref):
    qb = q_ref[0]
    kb = k_ref[0]
    logits = jax.lax.dot_general(qb, kb, (((1,), (1,)), ((), ())),
                                 preferred_element_type=jnp.float32)
    logits = logits * (1.0 / math.sqrt(D))
    mx = jnp.max(logits, axis=1, keepdims=True)
    e = jnp.exp(logits - mx)
    a = e / jnp.sum(e, axis=1, keepdims=True)
    attn_ref[0, :, :] = a
    vb = v_ref[0]
    contrib = jax.lax.dot_general(a.astype(bf16), vb, (((0,), (0,)), ((), ())),
                                  preferred_element_type=jnp.float32)
    val_ref[0, :, :] = contrib


def _out_kernel(val_ref, wp_ref, bp_ref, go_ref, bbo_ref, o_ref):
    vals = val_ref[...]
    acc = jnp.zeros((C, IN), jnp.float32)
    for h in range(H):
        acc += jnp.dot(vals[h], wp_ref[pl.ds(h * D, D), :],
                       preferred_element_type=jnp.float32)
    pre = acc + bp_ref[...]
    o_ref[...] = _ln(pre, go_ref[...], bbo_ref[...])


def kernel(x, W_ih, b_ih, W_k, b_k, g_k, bb_k, W_q, b_q, g_q, bb_q,
           W_v, b_v, g_v, bb_v, W_p, b_p, g_o, bb_o):
    f32 = jnp.float32
    b_ih2 = b_ih.reshape(1, C)
    b_q2, g_q2, bb_q2 = b_q.reshape(1, ED), g_q.reshape(1, ED), bb_q.reshape(1, ED)
    b_v2, g_v2, bb_v2 = b_v.reshape(1, ED), g_v.reshape(1, ED), bb_v.reshape(1, ED)
    b_k2, g_k2, bb_k2 = b_k.reshape(1, ED), g_k.reshape(1, ED), bb_k.reshape(1, ED)
    b_p2, g_o2, bb_o2 = b_p.reshape(1, IN), g_o.reshape(1, IN), bb_o.reshape(1, IN)

    full = lambda shape: pl.BlockSpec(shape, lambda *_: tuple(0 for _ in shape))

    nsb = S // SB
    c, q_, v_, topk_idx = pl.pallas_call(
        _proj_kernel,
        grid=(nsb,),
        in_specs=[
            pl.BlockSpec((SB, IN), lambda i: (i, 0)),
            full((IN, C)), full((1, C)),
            full((IN, ED)), full((1, ED)), full((1, ED)), full((1, ED)),
            full((IN, ED)), full((1, ED)), full((1, ED)), full((1, ED)),
        ],
        out_specs=[
            pl.BlockSpec((SB, C), lambda i: (i, 0)),
            pl.BlockSpec((H, SB, D), lambda i: (0, i, 0)),
            pl.BlockSpec((H, SB, D), lambda i: (0, i, 0)),
            full((K, C)),
        ],
        out_shape=[
            jax.ShapeDtypeStruct((S, C), f32),
            jax.ShapeDtypeStruct((H, S, D), bf16),
            jax.ShapeDtypeStruct((H, S, D), bf16),
            jax.ShapeDtypeStruct((K, C), jnp.int32),
        ],
        scratch_shapes=[pltpu.VMEM((S, C), f32)],
    )(x, W_ih, b_ih2, W_q, b_q2, g_q2, bb_q2, W_v, b_v2, g_v2, bb_v2)

    k_ = pl.pallas_call(
        _kproj_kernel,
        grid=(K,),
        in_specs=[
            full((K, C)),
            full((S, IN)),
            pl.BlockSpec((IN, ED), lambda j: (j, 0)),
            full((1, ED)), full((1, ED)), full((1, ED)),
        ],
        out_specs=pl.BlockSpec((H, C, D), lambda j: (0, 0, 0)),
        out_shape=jax.ShapeDtypeStruct((H, C, D), bf16),
        scratch_shapes=[pltpu.VMEM((C, ED), f32)],
    )(topk_idx, x, W_k, b_k2, g_k2, bb_k2)

    attn, values = pl.pallas_call(
        _attn_kernel,
        grid=(H,),
        in_specs=[
            pl.BlockSpec((1, S, D), lambda h: (h, 0, 0)),
            pl.BlockSpec((1, C, D), lambda h: (h, 0, 0)),
            pl.BlockSpec((1, S, D), lambda h: (h, 0, 0)),
        ],
        out_specs=[
            pl.BlockSpec((1, S, C), lambda h: (h, 0, 0)),
            pl.BlockSpec((1, C, D), lambda h: (h, 0, 0)),
        ],
        out_shape=[
            jax.ShapeDtypeStruct((H, S, C), f32),
            jax.ShapeDtypeStruct((H, C, D), f32),
        ],
    )(q_, k_, v_)

    o = pl.pallas_call(
        _out_kernel,
        out_shape=jax.ShapeDtypeStruct((C, IN), f32),
    )(values, W_p, b_p2, g_o2, bb_o2)

    return (o, c, attn, topk_idx)


# speedup vs baseline: 1.3562x; 1.0580x over previous
import functools
import math

import jax
import jax.numpy as jnp
from jax.experimental import pallas as pl
from jax.experimental.pallas import tpu as pltpu

S = 4096
IN = 1024
H = 16
D = 64
ED = H * D
C = 100
K = 8
SB = 1024
HB = 2

bf16 = jnp.bfloat16


def _ln(x, g, b, eps=1e-5):
    m = jnp.mean(x, axis=-1, keepdims=True)
    d = x - m
    v = jnp.mean(d * d, axis=-1, keepdims=True)
    return d * jax.lax.rsqrt(v + eps) * g + b


def _bdot(a, b):
    return jnp.dot(a.astype(bf16), b.astype(bf16), preferred_element_type=jnp.float32)


def _proj_kernel(x_ref, wih_ref, bih_ref, wq_ref, bq_ref, gq_ref, bbq_ref,
                 wv_ref, bv_ref, gv_ref, bbv_ref,
                 c_ref, q_ref, v_ref, idx_ref, cacc_ref):
    i = pl.program_id(0)
    xb = x_ref[...]
    cb = jnp.dot(xb, wih_ref[...], preferred_element_type=jnp.float32) + bih_ref[...]
    c_ref[...] = cb
    cacc_ref[pl.ds(i * SB, SB), :] = cb
    qp = _bdot(xb, wq_ref[...]) + bq_ref[...]
    qln = _ln(qp, gq_ref[...], bbq_ref[...]).astype(bf16)
    q_ref[...] = jnp.transpose(qln.reshape(SB, H, D), (1, 0, 2))
    vp = _bdot(xb, wv_ref[...]) + bv_ref[...]
    vln = _ln(vp, gv_ref[...], bbv_ref[...]).astype(bf16)
    v_ref[...] = jnp.transpose(vln.reshape(SB, H, D), (1, 0, 2))

    @pl.when(i == (S // SB) - 1)
    def _():
        cv = cacc_ref[...]
        iota = jax.lax.broadcasted_iota(jnp.int32, (S, C), 0)
        for k in range(K):
            m = jnp.max(cv, axis=0, keepdims=True)
            idx = jnp.min(jnp.where(cv >= m, iota, S), axis=0)
            idx_ref[k, :] = idx
            cv = jnp.where(iota == idx[None, :], -jnp.inf, cv)


def _kproj_kernel(idx_ref, x_ref, wk_ref, bk_ref, gk_ref, bbk_ref, k_ref, acc_ref):
    j = pl.program_id(0)
    idx_j = idx_ref[pl.ds(j, 1), :]
    onehot = (jax.lax.broadcasted_iota(jnp.int32, (C, S), 1)
              == jnp.reshape(idx_j, (C, 1))).astype(bf16)
    g = jnp.dot(onehot, x_ref[...].astype(bf16),
                preferred_element_type=jnp.float32)
    contrib = _bdot(g, wk_ref[...])

    @pl.when(j == 0)
    def _():
        acc_ref[...] = contrib

    @pl.when(j > 0)
    def _():
        acc_ref[...] += contrib

    @pl.when(j == K - 1)
    def _():
        pre = acc_ref[...] + bk_ref[...]
        kf = _ln(pre, gk_ref[...], bbk_ref[...]).astype(bf16)
        k_ref[...] = jnp.transpose(kf.reshape(C, H, D), (1, 0, 2))


def _attn_kernel(q_ref, k_ref, v_ref, attn_ref, val_ref):
    for h in range(HB):
        qb = q_ref[h]
        kb = k_ref[h]
        logits = jax.lax.dot_general(qb, kb, (((1,), (1,)), ((), ())),
                                     preferred_element_type=jnp.float32)
        logits = logits * (1.0 / math.sqrt(D))
        e = jnp.exp(logits)
        a = e / jnp.sum(e, axis=1, keepdims=True)
        attn_ref[h, :, :] = a
        vb = v_ref[h]
        contrib = jax.lax.dot_general(a.astype(bf16), vb, (((0,), (0,)), ((), ())),
                                      preferred_element_type=jnp.float32)
        val_ref[h, :, :] = contrib


def _out_kernel(val_ref, wp_ref, bp_ref, go_ref, bbo_ref, o_ref):
    vals = val_ref[...]
    acc = jnp.zeros((C, IN), jnp.float32)
    for h in range(H):
        acc += jnp.dot(vals[h], wp_ref[pl.ds(h * D, D), :],
                       preferred_element_type=jnp.float32)
    pre = acc + bp_ref[...]
    o_ref[...] = _ln(pre, go_ref[...], bbo_ref[...])


def kernel(x, W_ih, b_ih, W_k, b_k, g_k, bb_k, W_q, b_q, g_q, bb_q,
           W_v, b_v, g_v, bb_v, W_p, b_p, g_o, bb_o):
    f32 = jnp.float32
    b_ih2 = b_ih.reshape(1, C)
    b_q2, g_q2, bb_q2 = b_q.reshape(1, ED), g_q.reshape(1, ED), bb_q.reshape(1, ED)
    b_v2, g_v2, bb_v2 = b_v.reshape(1, ED), g_v.reshape(1, ED), bb_v.reshape(1, ED)
    b_k2, g_k2, bb_k2 = b_k.reshape(1, ED), g_k.reshape(1, ED), bb_k.reshape(1, ED)
    b_p2, g_o2, bb_o2 = b_p.reshape(1, IN), g_o.reshape(1, IN), bb_o.reshape(1, IN)

    full = lambda shape: pl.BlockSpec(shape, lambda *_: tuple(0 for _ in shape))

    nsb = S // SB
    c, q_, v_, topk_idx = pl.pallas_call(
        _proj_kernel,
        grid=(nsb,),
        in_specs=[
            pl.BlockSpec((SB, IN), lambda i: (i, 0)),
            full((IN, C)), full((1, C)),
            full((IN, ED)), full((1, ED)), full((1, ED)), full((1, ED)),
            full((IN, ED)), full((1, ED)), full((1, ED)), full((1, ED)),
        ],
        out_specs=[
            pl.BlockSpec((SB, C), lambda i: (i, 0)),
            pl.BlockSpec((H, SB, D), lambda i: (0, i, 0)),
            pl.BlockSpec((H, SB, D), lambda i: (0, i, 0)),
            full((K, C)),
        ],
        out_shape=[
            jax.ShapeDtypeStruct((S, C), f32),
            jax.ShapeDtypeStruct((H, S, D), bf16),
            jax.ShapeDtypeStruct((H, S, D), bf16),
            jax.ShapeDtypeStruct((K, C), jnp.int32),
        ],
        scratch_shapes=[pltpu.VMEM((S, C), f32)],
    )(x, W_ih, b_ih2, W_q, b_q2, g_q2, bb_q2, W_v, b_v2, g_v2, bb_v2)

    k_ = pl.pallas_call(
        _kproj_kernel,
        grid=(K,),
        in_specs=[
            full((K, C)),
            full((S, IN)),
            pl.BlockSpec((IN, ED), lambda j: (j, 0)),
            full((1, ED)), full((1, ED)), full((1, ED)),
        ],
        out_specs=pl.BlockSpec((H, C, D), lambda j: (0, 0, 0)),
        out_shape=jax.ShapeDtypeStruct((H, C, D), bf16),
        scratch_shapes=[pltpu.VMEM((C, ED), f32)],
    )(topk_idx, x, W_k, b_k2, g_k2, bb_k2)

    attn, values = pl.pallas_call(
        _attn_kernel,
        grid=(H // HB,),
        in_specs=[
            pl.BlockSpec((HB, S, D), lambda h: (h, 0, 0)),
            pl.BlockSpec((HB, C, D), lambda h: (h, 0, 0)),
            pl.BlockSpec((HB, S, D), lambda h: (h, 0, 0)),
        ],
        out_specs=[
            pl.BlockSpec((HB, S, C), lambda h: (h, 0, 0)),
            pl.BlockSpec((HB, C, D), lambda h: (h, 0, 0)),
        ],
        out_shape=[
            jax.ShapeDtypeStruct((H, S, C), f32),
            jax.ShapeDtypeStruct((H, C, D), f32),
        ],
    )(q_, k_, v_)

    o = pl.pallas_call(
        _out_kernel,
        out_shape=jax.ShapeDtypeStruct((C, IN), f32),
    )(values, W_p, b_p2, g_o2, bb_o2)

    return (o, c, attn, topk_idx)
